# Initial kernel scaffold; baseline (speedup 1.0000x reference)
import jax
import jax.numpy as jnp
from jax.experimental import pallas as pl


def kernel(x, edge_index, c_param, W_enc, b_enc, W0, b0, W1, b1, W2, b2, W_head, b_head):
    raise NotImplementedError("write your pallas kernel here")


# speedup vs baseline: 2.7491x; 2.7491x over previous
import functools

import jax
import jax.numpy as jnp
from jax import lax
from jax.experimental import pallas as pl
from jax.experimental.pallas import tpu as pltpu
import jax.experimental.pallas.tpu_sc as plsc

N = 10000
E = 160000
HID = 256
D_OUT = 128
EPS = 1e-7

NPAD = 10240
TRASH = NPAD - 1
B = 128
EP_TILE = NPAD
KCH = EP_TILE // B
E_PAD = 16 * EP_TILE
RB = 400
NB = N // RB

_mesh = plsc.VectorSubcoreMesh(core_axis_name="c", subcore_axis_name="s")


@functools.partial(
    pl.kernel,
    out_type=jax.ShapeDtypeStruct((2, NPAD, 128), jnp.float32),
    mesh=_mesh,
    scratch_types=[
        pltpu.VMEM((KCH, B), jnp.int32),
        pltpu.VMEM((KCH, B), jnp.int32),
        pltpu.VMEM((B, 128), jnp.float32),
        pltpu.VMEM_SHARED((NPAD, 128), jnp.float32),
        pltpu.SemaphoreType.DMA,
    ],
)
def _sc_agg(m_hbm, src2_hbm, dst2_hbm, zeros_hbm, out_hbm,
            src_v, dst_v, rows_v, agg_sh, sem):
    cc = lax.axis_index("c")
    s = lax.axis_index("s")
    pltpu.sync_copy(src2_hbm.at[cc, pl.ds(s * KCH, KCH)], src_v)
    pltpu.sync_copy(dst2_hbm.at[pl.ds(s * KCH, KCH)], dst_v)
    pltpu.sync_copy(zeros_hbm, agg_sh.at[pl.ds(s * (NPAD // 16), NPAD // 16)])
    plsc.subcore_barrier()

    def chunk(j, carry):
        pltpu.async_copy(m_hbm.at[src_v.at[j]], rows_v, sem).wait()
        pltpu.sync_copy(rows_v, agg_sh.at[dst_v.at[j]], add=True)
        return carry

    lax.fori_loop(0, KCH, chunk, 0)
    plsc.subcore_barrier()
    nr = NPAD // 16
    pltpu.sync_copy(agg_sh.at[pl.ds(s * nr, nr)],
                    out_hbm.at[cc, pl.ds(s * nr, nr)])


@functools.partial(
    pl.kernel,
    out_type=jax.ShapeDtypeStruct((NPAD, 128), jnp.float32),
    mesh=_mesh,
    scratch_types=[
        pltpu.VMEM((KCH, B), jnp.int32),
        pltpu.VMEM((B, 128), jnp.float32),
        pltpu.VMEM_SHARED((NPAD, 128), jnp.float32),
        pltpu.SemaphoreType.DMA,
    ],
)
def _sc_deg(dst2_hbm, ones_hbm, zeros_hbm, out_hbm,
            dst_v, ones_v, deg_sh, sem):
    cc = lax.axis_index("c")
    s = lax.axis_index("s")
    pltpu.sync_copy(dst2_hbm.at[pl.ds(s * KCH, KCH)], dst_v)
    pltpu.sync_copy(ones_hbm, ones_v)
    pltpu.sync_copy(zeros_hbm, deg_sh.at[pl.ds(s * (NPAD // 16), NPAD // 16)])
    plsc.subcore_barrier()

    def chunk(j, carry):
        pltpu.sync_copy(ones_v, deg_sh.at[dst_v.at[j]], add=True)
        return carry

    lax.fori_loop(0, KCH, chunk, 0)
    plsc.subcore_barrier()
    nr = NPAD // 16

    @pl.when(cc == 0)
    def _():
        pltpu.sync_copy(deg_sh.at[pl.ds(s * nr, nr)],
                        out_hbm.at[pl.ds(s * nr, nr)])


def _row_norm(v):
    n = jnp.sqrt(jnp.sum(v * v, axis=-1, keepdims=True))
    return jnp.maximum(n, EPS)


def _exp_map(v, sc):
    n = _row_norm(v)
    return jnp.tanh(sc * n) * v / (sc * n)


def _log_map(y, sc):
    n = _row_norm(y)
    scn = jnp.clip(sc * n, EPS, 1.0 - 1e-5)
    atan = 0.5 * jnp.log((1.0 + scn) / (1.0 - scn))
    return atan * y / (sc * n)


def _tc_enc_body(x_ref, we_ref, be_ref, w_ref, b_ref, c_ref, out_ref):
    sc = jnp.sqrt(c_ref[0, 0])
    t = jnp.dot(x_ref[...], we_ref[...],
                preferred_element_type=jnp.float32) + be_ref[...]
    ht = _log_map(_exp_map(t, sc), sc)
    m = jnp.dot(ht, w_ref[...], preferred_element_type=jnp.float32) + b_ref[...]
    out_ref[0] = m[:, :128]
    out_ref[1] = m[:, 128:]


def _tc_enc(x, w_enc, b_enc, w0, b0, c2d):
    return pl.pallas_call(
        _tc_enc_body,
        grid=(NB,),
        in_specs=[
            pl.BlockSpec((RB, HID), lambda i: (i, 0)),
            pl.BlockSpec((HID, HID), lambda i: (0, 0)),
            pl.BlockSpec((1, HID), lambda i: (0, 0)),
            pl.BlockSpec((HID, HID), lambda i: (0, 0)),
            pl.BlockSpec((1, HID), lambda i: (0, 0)),
            pl.BlockSpec((1, 1), lambda i: (0, 0)),
        ],
        out_specs=pl.BlockSpec((2, RB, 128), lambda i: (0, i, 0)),
        out_shape=jax.ShapeDtypeStruct((2, N, 128), jnp.float32),
    )(x, w_enc, b_enc, w0, b0, c2d)


def _make_tc_mid_body(nh):
    def body(agg_ref, deg_ref, w_ref, b_ref, c_ref, out_ref):
        sc = jnp.sqrt(c_ref[0, 0])
        a = jnp.concatenate([agg_ref[0], agg_ref[1]], axis=1)
        d = jnp.maximum(deg_ref[:, 0:1], 1.0)
        a = a / d
        h = _exp_map(a, sc)
        h = _exp_map(_log_map(h, sc), sc)
        ht = _log_map(h, sc)
        m = jnp.dot(ht, w_ref[...],
                    preferred_element_type=jnp.float32) + b_ref[...]
        for k in range(nh):
            out_ref[k] = m[:, k * 128:(k + 1) * 128]
    return body


def _tc_mid(agg, deg, w, b, c2d, nh):
    return pl.pallas_call(
        _make_tc_mid_body(nh),
        grid=(NB,),
        in_specs=[
            pl.BlockSpec((2, RB, 128), lambda i: (0, i, 0)),
            pl.BlockSpec((RB, 128), lambda i: (i, 0)),
            pl.BlockSpec((HID, nh * 128), lambda i: (0, 0)),
            pl.BlockSpec((1, nh * 128), lambda i: (0, 0)),
            pl.BlockSpec((1, 1), lambda i: (0, 0)),
        ],
        out_specs=pl.BlockSpec((nh, RB, 128), lambda i: (0, i, 0)),
        out_shape=jax.ShapeDtypeStruct((nh, N, 128), jnp.float32),
    )(agg, deg, w, b, c2d)


@jax.jit
def kernel(x, edge_index, c_param, W_enc, b_enc, W0, b0, W1, b1, W2, b2,
           W_head, b_head):
    c2d = (jnp.abs(c_param) + 1e-5).reshape(1, 1).astype(jnp.float32)
    ei = edge_index.astype(jnp.int32)
    src = ei[0]
    dst = ei[1]
    npad = E_PAD - E
    src_pad = jnp.concatenate([src, jnp.zeros((npad,), jnp.int32)])
    dst_pad = jnp.concatenate([dst, jnp.full((npad,), TRASH, jnp.int32)])
    src2 = jnp.stack([src_pad, src_pad + N]).reshape(2, E_PAD // B, B)
    dst2 = dst_pad.reshape(E_PAD // B, B)
    zeros128 = jnp.zeros((NPAD // 16, 128), jnp.float32)
    ones128 = jnp.ones((B, 128), jnp.float32)

    deg = _sc_deg(dst2, ones128, zeros128)

    m = _tc_enc(x, W_enc, b_enc.reshape(1, -1), W0, b0.reshape(1, -1), c2d)
    agg = _sc_agg(m.reshape(2 * N, 128), src2, dst2, zeros128)
    m = _tc_mid(agg, deg, W1, b1.reshape(1, -1), c2d, nh=2)
    agg = _sc_agg(m.reshape(2 * N, 128), src2, dst2, zeros128)
    m = _tc_mid(agg, deg, W2, b2.reshape(1, -1), c2d, nh=2)
    agg = _sc_agg(m.reshape(2 * N, 128), src2, dst2, zeros128)
    out = _tc_mid(agg, deg, W_head, b_head.reshape(1, -1), c2d, nh=1)
    return out[0]


# speedup vs baseline: 3.0563x; 1.1117x over previous
import functools

import jax
import jax.numpy as jnp
from jax import lax
from jax.experimental import pallas as pl
from jax.experimental.pallas import tpu as pltpu
import jax.experimental.pallas.tpu_sc as plsc

N = 10000
E = 160000
HID = 256
D_OUT = 128
EPS = 1e-7

NPAD = 10240
TRASH = NPAD - 1
B = 128
EP_TILE = NPAD
KCH = EP_TILE // B
E_PAD = 16 * EP_TILE
RB = 400
NB = N // RB

_mesh = plsc.VectorSubcoreMesh(core_axis_name="c", subcore_axis_name="s")


@functools.partial(
    pl.kernel,
    out_type=jax.ShapeDtypeStruct((2, NPAD, 128), jnp.float32),
    mesh=_mesh,
    scratch_types=[
        pltpu.VMEM((KCH, B), jnp.int32),
        pltpu.VMEM((B,), jnp.int32),
        pltpu.VMEM((B,), jnp.int32),
        pltpu.VMEM((B,), jnp.int32),
        pltpu.VMEM((B,), jnp.int32),
        pltpu.VMEM((B, 128), jnp.float32),
        pltpu.VMEM((B, 128), jnp.float32),
        pltpu.VMEM_SHARED((NPAD, 128), jnp.float32),
        pltpu.SemaphoreType.DMA,
        pltpu.SemaphoreType.DMA,
    ],
)
def _sc_agg(m_hbm, packed_hbm, zeros_hbm, out_hbm,
            packed_v, sidx0_v, didx0_v, sidx1_v, didx1_v,
            rows0_v, rows1_v, agg_sh, sem0, sem1):
    cc = lax.axis_index("c")
    s = lax.axis_index("s")
    pltpu.sync_copy(packed_hbm.at[pl.ds(s * KCH, KCH)], packed_v)
    pltpu.sync_copy(zeros_hbm, agg_sh.at[pl.ds(s * (NPAD // 16), NPAD // 16)])
    plsc.subcore_barrier()

    row_off = cc * N

    def unpack(j, sidx, didx):
        for k in range(B // 16):
            p = packed_v[j, pl.ds(k * 16, 16)]
            sidx[pl.ds(k * 16, 16)] = (p & 0xFFFF) + row_off
            didx[pl.ds(k * 16, 16)] = lax.shift_right_logical(p, 16)

    def gather(sidx, buf, sem):
        return pltpu.make_async_copy(m_hbm.at[sidx], buf, sem)

    unpack(0, sidx0_v, didx0_v)
    gather(sidx0_v, rows0_v, sem0).start()

    def pair(i, carry):
        j1 = 2 * i + 1
        unpack(j1, sidx1_v, didx1_v)
        gather(sidx0_v, rows0_v, sem0).wait()
        gather(sidx1_v, rows1_v, sem1).start()
        pltpu.sync_copy(rows0_v, agg_sh.at[didx0_v], add=True)

        @pl.when(i + 1 < KCH // 2)
        def _():
            unpack(j1 + 1, sidx0_v, didx0_v)

        gather(sidx1_v, rows1_v, sem1).wait()

        @pl.when(i + 1 < KCH // 2)
        def _():
            gather(sidx0_v, rows0_v, sem0).start()

        pltpu.sync_copy(rows1_v, agg_sh.at[didx1_v], add=True)
        return carry

    lax.fori_loop(0, KCH // 2, pair, 0)
    plsc.subcore_barrier()
    nr = NPAD // 16
    pltpu.sync_copy(agg_sh.at[pl.ds(s * nr, nr)],
                    out_hbm.at[cc, pl.ds(s * nr, nr)])


@functools.partial(
    pl.kernel,
    out_type=jax.ShapeDtypeStruct((NPAD, 128), jnp.float32),
    mesh=_mesh,
    scratch_types=[
        pltpu.VMEM((KCH, B), jnp.int32),
        pltpu.VMEM((B, 128), jnp.float32),
        pltpu.VMEM_SHARED((NPAD, 128), jnp.float32),
        pltpu.SemaphoreType.DMA,
    ],
)
def _sc_deg(dst2_hbm, ones_hbm, zeros_hbm, out_hbm,
            dst_v, ones_v, deg_sh, sem):
    cc = lax.axis_index("c")
    s = lax.axis_index("s")
    pltpu.sync_copy(dst2_hbm.at[pl.ds(s * KCH, KCH)], dst_v)
    pltpu.sync_copy(ones_hbm, ones_v)
    pltpu.sync_copy(zeros_hbm, deg_sh.at[pl.ds(s * (NPAD // 16), NPAD // 16)])
    plsc.subcore_barrier()

    def chunk(j, carry):
        pltpu.sync_copy(ones_v, deg_sh.at[dst_v.at[j]], add=True)
        return carry

    lax.fori_loop(0, KCH, chunk, 0)
    plsc.subcore_barrier()
    nr = NPAD // 16

    @pl.when(cc == 0)
    def _():
        pltpu.sync_copy(deg_sh.at[pl.ds(s * nr, nr)],
                        out_hbm.at[pl.ds(s * nr, nr)])


def _row_norm(v):
    n = jnp.sqrt(jnp.sum(v * v, axis=-1, keepdims=True))
    return jnp.maximum(n, EPS)


def _exp_map(v, sc):
    n = _row_norm(v)
    return jnp.tanh(sc * n) * v / (sc * n)


def _log_map(y, sc):
    n = _row_norm(y)
    scn = jnp.clip(sc * n, EPS, 1.0 - 1e-5)
    atan = 0.5 * jnp.log((1.0 + scn) / (1.0 - scn))
    return atan * y / (sc * n)


def _tc_enc_body(x_ref, we_ref, be_ref, w_ref, b_ref, c_ref, out_ref):
    sc = jnp.sqrt(c_ref[0, 0])
    t = jnp.dot(x_ref[...], we_ref[...],
                preferred_element_type=jnp.float32) + be_ref[...]
    ht = _log_map(_exp_map(t, sc), sc)
    m = jnp.dot(ht, w_ref[...], preferred_element_type=jnp.float32) + b_ref[...]
    out_ref[0] = m[:, :128]
    out_ref[1] = m[:, 128:]


def _tc_enc(x, w_enc, b_enc, w0, b0, c2d):
    return pl.pallas_call(
        _tc_enc_body,
        grid=(NB,),
        in_specs=[
            pl.BlockSpec((RB, HID), lambda i: (i, 0)),
            pl.BlockSpec((HID, HID), lambda i: (0, 0)),
            pl.BlockSpec((1, HID), lambda i: (0, 0)),
            pl.BlockSpec((HID, HID), lambda i: (0, 0)),
            pl.BlockSpec((1, HID), lambda i: (0, 0)),
            pl.BlockSpec((1, 1), lambda i: (0, 0)),
        ],
        out_specs=pl.BlockSpec((2, RB, 128), lambda i: (0, i, 0)),
        out_shape=jax.ShapeDtypeStruct((2, N, 128), jnp.float32),
    )(x, w_enc, b_enc, w0, b0, c2d)


def _make_tc_mid_body(nh):
    def body(agg_ref, deg_ref, w_ref, b_ref, c_ref, out_ref):
        sc = jnp.sqrt(c_ref[0, 0])
        a = jnp.concatenate([agg_ref[0], agg_ref[1]], axis=1)
        d = jnp.maximum(deg_ref[:, 0:1], 1.0)
        a = a / d
        h = _exp_map(a, sc)
        h = _exp_map(_log_map(h, sc), sc)
        ht = _log_map(h, sc)
        m = jnp.dot(ht, w_ref[...],
                    preferred_element_type=jnp.float32) + b_ref[...]
        for k in range(nh):
            out_ref[k] = m[:, k * 128:(k + 1) * 128]
    return body


def _tc_mid(agg, deg, w, b, c2d, nh):
    return pl.pallas_call(
        _make_tc_mid_body(nh),
        grid=(NB,),
        in_specs=[
            pl.BlockSpec((2, RB, 128), lambda i: (0, i, 0)),
            pl.BlockSpec((RB, 128), lambda i: (i, 0)),
            pl.BlockSpec((HID, nh * 128), lambda i: (0, 0)),
            pl.BlockSpec((1, nh * 128), lambda i: (0, 0)),
            pl.BlockSpec((1, 1), lambda i: (0, 0)),
        ],
        out_specs=pl.BlockSpec((nh, RB, 128), lambda i: (0, i, 0)),
        out_shape=jax.ShapeDtypeStruct((nh, N, 128), jnp.float32),
    )(agg, deg, w, b, c2d)


@jax.jit
def kernel(x, edge_index, c_param, W_enc, b_enc, W0, b0, W1, b1, W2, b2,
           W_head, b_head):
    c2d = (jnp.abs(c_param) + 1e-5).reshape(1, 1).astype(jnp.float32)
    ei = edge_index.astype(jnp.int32)
    src = ei[0]
    dst = ei[1]
    npad = E_PAD - E
    src_pad = jnp.concatenate([src, jnp.zeros((npad,), jnp.int32)])
    dst_pad = jnp.concatenate([dst, jnp.full((npad,), TRASH, jnp.int32)])
    packed = (src_pad | (dst_pad << 16)).reshape(E_PAD // B, B)
    dst2 = dst_pad.reshape(E_PAD // B, B)
    zeros128 = jnp.zeros((NPAD // 16, 128), jnp.float32)
    ones128 = jnp.ones((B, 128), jnp.float32)

    deg = _sc_deg(dst2, ones128, zeros128)

    m = _tc_enc(x, W_enc, b_enc.reshape(1, -1), W0, b0.reshape(1, -1), c2d)
    agg = _sc_agg(m.reshape(2 * N, 128), packed, zeros128)
    m = _tc_mid(agg, deg, W1, b1.reshape(1, -1), c2d, nh=2)
    agg = _sc_agg(m.reshape(2 * N, 128), packed, zeros128)
    m = _tc_mid(agg, deg, W2, b2.reshape(1, -1), c2d, nh=2)
    agg = _sc_agg(m.reshape(2 * N, 128), packed, zeros128)
    out = _tc_mid(agg, deg, W_head, b_head.reshape(1, -1), c2d, nh=1)
    return out[0]
